# baseline (speedup 1.0000x reference)
import jax
import jax.numpy as jnp
from jax.experimental import pallas as pl

N = 1024
C_IN = 128
C_OUT = 64
ALPHA = 0.2
LOG2E = 1.4426950408889634


def _gat_kernel(x_ref, adj_ref, w_ref, b_ref, a_ref, o_ref):
    x = x_ref[...]
    w = w_ref[...]
    h = jax.lax.dot_general(x, w, (((1,), (1,)), ((), ())),
                            preferred_element_type=jnp.float32) + b_ref[...]
    a = a_ref[...]
    s_col = jax.lax.dot_general(h, a[:, :C_OUT], (((1,), (1,)), ((), ())),
                                preferred_element_type=jnp.float32)
    t_row = jax.lax.dot_general(a[:, C_OUT:], h, (((1,), (1,)), ((), ())),
                                preferred_element_type=jnp.float32)
    tmax = jnp.max(t_row, axis=1, keepdims=True)
    sm = s_col + tmax
    m = jnp.maximum(sm, ALPHA * sm)

    s1 = (s_col - m) * LOG2E
    s2 = (ALPHA * s_col - m) * LOG2E
    t1 = t_row * LOG2E
    t2 = t_row * (ALPHA * LOG2E)

    arg = jnp.maximum(s1 + t1, s2 + t2)
    e = jnp.where(adj_ref[...] != 0, jnp.exp2(arg), 0.0)

    lane = jax.lax.broadcasted_iota(jnp.int32, (N, C_OUT), 1)
    ones_blk = jnp.where(lane == 0, 1.0, 0.0)
    h_ext = jnp.concatenate([h, ones_blk], axis=1)
    acc = jax.lax.dot_general(e, h_ext, (((1,), (0,)), ((), ())),
                              preferred_element_type=jnp.float32)
    ssum = acc[:, C_OUT:C_OUT + 1]
    recip = 1.0 / jnp.where(ssum > 0, ssum, 1.0)
    hmean = jnp.sum(h, axis=0, keepdims=True) * (1.0 / N)
    o_ref[...] = jnp.where(ssum > 0, acc[:, :C_OUT] * recip,
                           jnp.broadcast_to(hmean, (N, C_OUT)))


def kernel(node_feats_in, adj_matrix, W, b, a):
    x = node_feats_in.reshape(N, C_IN)
    adj = adj_matrix.reshape(N, N)
    b2 = b.reshape(1, C_OUT)
    out = pl.pallas_call(
        _gat_kernel,
        out_shape=jax.ShapeDtypeStruct((N, C_OUT), jnp.float32),
    )(x, adj, W, b2, a)
    return out.reshape(1, N, C_OUT)


# speedup vs baseline: 1.0135x; 1.0135x over previous
import jax
import jax.numpy as jnp
from jax.experimental import pallas as pl

N = 1024
C_IN = 128
C_OUT = 64
ALPHA = 0.2
LOG2E = 1.4426950408889634


def _gat_kernel(x_ref, adj_ref, w_ref, b_ref, a_ref, o_ref):
    x = x_ref[...]
    w = w_ref[...]
    h = jax.lax.dot_general(x, w, (((1,), (1,)), ((), ())),
                            preferred_element_type=jnp.float32) + b_ref[...]
    a = a_ref[...]
    s_col = jax.lax.dot_general(h, a[:, :C_OUT], (((1,), (1,)), ((), ())),
                                preferred_element_type=jnp.float32)
    t_row = jax.lax.dot_general(a[:, C_OUT:], h, (((1,), (1,)), ((), ())),
                                preferred_element_type=jnp.float32)
    tmax = jnp.max(t_row, axis=1, keepdims=True)
    sm = s_col + tmax
    m = jnp.maximum(sm, ALPHA * sm)

    s1 = (s_col - m) * LOG2E
    s2 = (ALPHA * s_col - m) * LOG2E
    t1 = t_row * LOG2E
    t2 = t_row * (ALPHA * LOG2E)

    arg = jnp.maximum(s1 + t1, s2 + t2)
    e = jnp.where(adj_ref[...] != 0, jnp.exp2(arg), 0.0)

    lane = jax.lax.broadcasted_iota(jnp.int32, (N, C_OUT), 1)
    ones_blk = jnp.where(lane == 0, 1.0, 0.0)
    h_ext = jnp.concatenate([h, ones_blk], axis=1)
    acc = jax.lax.dot_general(e.astype(jnp.bfloat16), h_ext.astype(jnp.bfloat16),
                              (((1,), (0,)), ((), ())),
                              preferred_element_type=jnp.float32)
    ssum = acc[:, C_OUT:C_OUT + 1]
    recip = 1.0 / jnp.where(ssum > 0, ssum, 1.0)
    hmean = jnp.sum(h, axis=0, keepdims=True) * (1.0 / N)
    o_ref[...] = jnp.where(ssum > 0, acc[:, :C_OUT] * recip,
                           jnp.broadcast_to(hmean, (N, C_OUT)))


def kernel(node_feats_in, adj_matrix, W, b, a):
    x = node_feats_in.reshape(N, C_IN)
    adj = adj_matrix.reshape(N, N)
    b2 = b.reshape(1, C_OUT)
    out = pl.pallas_call(
        _gat_kernel,
        out_shape=jax.ShapeDtypeStruct((N, C_OUT), jnp.float32),
    )(x, adj, W, b2, a)
    return out.reshape(1, N, C_OUT)
